# baseline (speedup 1.0000x reference)
import functools

import jax
import jax.numpy as jnp
import numpy as np
from jax import lax
from jax.experimental import pallas as pl
from jax.experimental.pallas import tpu as pltpu
from jax.experimental.pallas import tpu_sc as plsc

VOCAB_SIZE = 100000
D_MODEL = 768
MAX_LEN = 4096
BATCH = 4
SEQ_LEN = 4096

_INFO = plsc.get_sparse_core_info()
_NC, _NS, _L = _INFO.num_cores, _INFO.num_subcores, _INFO.num_lanes
_NW = _NC * _NS
_ROWS = BATCH * SEQ_LEN
_STRIPE = SEQ_LEN // _NW
_CHUNK = 32
_NQ = _STRIPE // _CHUNK
_NT = _NQ * BATCH
_VECS = D_MODEL // _L
_PAIRS = D_MODEL // (2 * _L)
_PWORDS = D_MODEL // 2
_NBUF = 4


def _sinusoidal_pos_encoding(max_len, d_model):
    pos = np.arange(max_len, dtype=np.float32)[:, None]
    i = np.arange(0, d_model, 2, dtype=np.float32)[None, :]
    angle = pos / np.power(10000.0, i / d_model)
    enc = np.zeros((max_len, d_model), dtype=np.float32)
    enc[:, 0::2] = np.sin(angle)
    enc[:, 1::2] = np.cos(angle)
    return enc


def _pack_pos_bf16(enc):
    u = enc.astype(np.float32).view(np.uint32)
    bf = ((u + 0x7FFF + ((u >> 16) & 1)) >> 16).astype(np.uint32)
    s, d = enc.shape
    bf = bf.reshape(s, d // 32, 2, 16)
    packed = bf[:, :, 0, :] | (bf[:, :, 1, :] << np.uint32(16))
    return packed.reshape(s, d // 2).astype(np.int32)


_POS_PACKED = _pack_pos_bf16(_sinusoidal_pos_encoding(MAX_LEN, D_MODEL))
_HI_MASK = np.int32(np.uint32(0xFFFF0000).astype(np.int32))


def _sc_body(table, idx, pos, out, idx_all, p0, p1, g0, g1, g2, g3,
             gs0, gs1, gs2, gs3, gt0, gt1, gt2, gt3,
             os0, os1, os2, os3, ot0, ot1, ot2, ot3, ps0, ps1, isem):
    wid = lax.axis_index("s") * _NC + lax.axis_index("c")
    w_base = wid * _STRIPE
    pos_v = (p0, p1)
    gath = (g0, g1, g2, g3)
    gsem = (gs0, gs1, gs2, gs3)
    gsem2 = (gt0, gt1, gt2, gt3)
    osem = (os0, os1, os2, os3)
    osem2 = (ot0, ot1, ot2, ot3)
    psem = (ps0, ps1)

    pos_desc = [None, None]
    pos_desc[0] = pltpu.async_copy(pos.at[pl.ds(w_base, _CHUNK)], p0, ps0)
    idesc = [
        pltpu.async_copy(idx.at[b, pl.ds(w_base, _STRIPE)], idx_all.at[b],
                         isem)
        for b in range(BATCH)
    ]
    for d in idesc:
        d.wait()

    _H = _CHUNK // 2

    def start_gather(t):
        b, q = t % BATCH, t // BATCH
        d1 = pltpu.async_copy(
            table.at[idx_all.at[b, pl.ds(q * _CHUNK, _H)]],
            gath[t % _NBUF].at[pl.ds(0, _H)], gsem[t % _NBUF])
        d2 = pltpu.async_copy(
            table.at[idx_all.at[b, pl.ds(q * _CHUNK + _H, _H)]],
            gath[t % _NBUF].at[pl.ds(_H, _H)], gsem2[t % _NBUF])
        return (d1, d2)

    gdesc = [None] * _NT
    odesc = [None] * _NT
    gdesc[0] = start_gather(0)
    gdesc[1] = start_gather(1)

    for t in range(_NT):
        b, q = t % BATCH, t // BATCH
        if t >= 2:
            odesc[t - 2][0].wait()
            odesc[t - 2][1].wait()
        if t + 2 < _NT:
            gdesc[t + 2] = start_gather(t + 2)
        if t % BATCH == 0 and q + 1 < _NQ:
            pos_desc[(q + 1) % 2] = pltpu.async_copy(
                pos.at[pl.ds(w_base + (q + 1) * _CHUNK, _CHUNK)],
                pos_v[(q + 1) % 2], psem[(q + 1) % 2])
        gdesc[t][0].wait()
        gdesc[t][1].wait()
        if t % BATCH == 0:
            pos_desc[q % 2].wait()
        g = gath[t % _NBUF]
        p = pos_v[q % 2]

        @plsc.parallel_loop(0, _CHUNK, step=1, unroll=1)
        def row_add(r, g=g, p=p):
            for j in range(_PAIRS):
                w = p[r, pl.ds(j * _L, _L)]
                lo = lax.bitcast_convert_type(lax.shift_left(w, 16), jnp.float32)
                hi = lax.bitcast_convert_type(lax.bitwise_and(w, _HI_MASK), jnp.float32)
                plsc.addupdate(g.at[r, pl.ds(2 * j * _L, _L)], lo)
                plsc.addupdate(g.at[r, pl.ds((2 * j + 1) * _L, _L)], hi)

        obase = b * SEQ_LEN + w_base + q * _CHUNK
        odesc[t] = (
            pltpu.async_copy(g.at[pl.ds(0, _H)],
                             out.at[pl.ds(obase, _H)], osem[t % _NBUF]),
            pltpu.async_copy(g.at[pl.ds(_H, _H)],
                             out.at[pl.ds(obase + _H, _H)],
                             osem2[t % _NBUF]),
        )
    for d in odesc[_NT - 2] + odesc[_NT - 1]:
        d.wait()


@jax.jit
def _embed(idx, tok_table, pos_packed):
    mesh = plsc.VectorSubcoreMesh(core_axis_name="c", subcore_axis_name="s")
    run = functools.partial(
        pl.kernel,
        mesh=mesh,
        out_type=jax.ShapeDtypeStruct((_ROWS, D_MODEL), jnp.float32),
        scratch_types=(
            [pltpu.VMEM((BATCH, _STRIPE), jnp.int32)]
            + [pltpu.VMEM((_CHUNK, _PWORDS), jnp.int32)] * 2
            + [pltpu.VMEM((_CHUNK, D_MODEL), jnp.float32)] * _NBUF
            + [pltpu.SemaphoreType.DMA] * (4 * _NBUF + 3)
        ),
    )(_sc_body)
    return run(tok_table, idx, pos_packed)


def kernel(x, tok_table):
    idx = x.astype(jnp.int32)
    pos_packed = jnp.asarray(_POS_PACKED)
    out = _embed(idx, tok_table, pos_packed)
    return out.reshape(BATCH, SEQ_LEN, D_MODEL)


# speedup vs baseline: 1.0098x; 1.0098x over previous
import functools

import jax
import jax.numpy as jnp
import numpy as np
from jax import lax
from jax.experimental import pallas as pl
from jax.experimental.pallas import tpu as pltpu
from jax.experimental.pallas import tpu_sc as plsc

VOCAB_SIZE = 100000
D_MODEL = 768
MAX_LEN = 4096
BATCH = 4
SEQ_LEN = 4096

_INFO = plsc.get_sparse_core_info()
_NC, _NS, _L = _INFO.num_cores, _INFO.num_subcores, _INFO.num_lanes
_NW = _NC * _NS
_ROWS = BATCH * SEQ_LEN
_STRIPE = SEQ_LEN // _NW
_CHUNK = 32
_NQ = _STRIPE // _CHUNK
_NT = _NQ * BATCH
_VECS = D_MODEL // _L
_PAIRS = D_MODEL // (2 * _L)
_PWORDS = D_MODEL // 2
_NBUF = 4


def _sinusoidal_pos_encoding(max_len, d_model):
    pos = np.arange(max_len, dtype=np.float32)[:, None]
    i = np.arange(0, d_model, 2, dtype=np.float32)[None, :]
    angle = pos / np.power(10000.0, i / d_model)
    enc = np.zeros((max_len, d_model), dtype=np.float32)
    enc[:, 0::2] = np.sin(angle)
    enc[:, 1::2] = np.cos(angle)
    return enc


def _pack_pos_bf16(enc):
    u = enc.astype(np.float32).view(np.uint32)
    bf = ((u + 0x7FFF + ((u >> 16) & 1)) >> 16).astype(np.uint32)
    s, d = enc.shape
    bf = bf.reshape(s, d // 32, 2, 16)
    packed = bf[:, :, 0, :] | (bf[:, :, 1, :] << np.uint32(16))
    return packed.reshape(s, d // 2).astype(np.int32)


_POS_PACKED = _pack_pos_bf16(_sinusoidal_pos_encoding(MAX_LEN, D_MODEL))
_HI_MASK = np.int32(np.uint32(0xFFFF0000).astype(np.int32))


def _sc_body(table, idx, pos, out, idx_all, p0, p1, g0, g1, g2, g3,
             gs0, gs1, gs2, gs3, gt0, gt1, gt2, gt3,
             gu0, gu1, gu2, gu3, gv0, gv1, gv2, gv3,
             os0, os1, os2, os3, ps0, ps1, isem):
    wid = lax.axis_index("s") * _NC + lax.axis_index("c")
    w_base = wid * _STRIPE
    pos_v = (p0, p1)
    gath = (g0, g1, g2, g3)
    gsem = (gs0, gs1, gs2, gs3)
    gsem2 = (gt0, gt1, gt2, gt3)
    gsem3 = (gu0, gu1, gu2, gu3)
    gsem4 = (gv0, gv1, gv2, gv3)
    osem = (os0, os1, os2, os3)
    psem = (ps0, ps1)

    pos_desc = [None, None]
    pos_desc[0] = pltpu.async_copy(pos.at[pl.ds(w_base, _CHUNK)], p0, ps0)
    idesc = [
        pltpu.async_copy(idx.at[b, pl.ds(w_base, _STRIPE)], idx_all.at[b],
                         isem)
        for b in range(BATCH)
    ]
    for d in idesc:
        d.wait()

    _H = _CHUNK // 4

    def start_gather(t):
        b, q = t % BATCH, t // BATCH
        sems = (gsem, gsem2, gsem3, gsem4)
        return tuple(
            pltpu.async_copy(
                table.at[idx_all.at[b, pl.ds(q * _CHUNK + i * _H, _H)]],
                gath[t % _NBUF].at[pl.ds(i * _H, _H)], sems[i][t % _NBUF])
            for i in range(4))

    gdesc = [None] * _NT
    odesc = [None] * _NT
    gdesc[0] = start_gather(0)
    gdesc[1] = start_gather(1)

    for t in range(_NT):
        b, q = t % BATCH, t // BATCH
        if t >= 2:
            odesc[t - 2].wait()
        if t + 2 < _NT:
            gdesc[t + 2] = start_gather(t + 2)
        if t % BATCH == 0 and q + 1 < _NQ:
            pos_desc[(q + 1) % 2] = pltpu.async_copy(
                pos.at[pl.ds(w_base + (q + 1) * _CHUNK, _CHUNK)],
                pos_v[(q + 1) % 2], psem[(q + 1) % 2])
        for d in gdesc[t]:
            d.wait()
        if t % BATCH == 0:
            pos_desc[q % 2].wait()
        g = gath[t % _NBUF]
        p = pos_v[q % 2]

        @plsc.parallel_loop(0, _CHUNK, step=1, unroll=1)
        def row_add(r, g=g, p=p):
            for j in range(_PAIRS):
                w = p[r, pl.ds(j * _L, _L)]
                lo = lax.bitcast_convert_type(lax.shift_left(w, 16), jnp.float32)
                hi = lax.bitcast_convert_type(lax.bitwise_and(w, _HI_MASK), jnp.float32)
                plsc.addupdate(g.at[r, pl.ds(2 * j * _L, _L)], lo)
                plsc.addupdate(g.at[r, pl.ds((2 * j + 1) * _L, _L)], hi)

        odesc[t] = pltpu.async_copy(
            g, out.at[pl.ds(b * SEQ_LEN + w_base + q * _CHUNK, _CHUNK)],
            osem[t % _NBUF])
    odesc[_NT - 2].wait()
    odesc[_NT - 1].wait()


@jax.jit
def _embed(idx, tok_table, pos_packed):
    mesh = plsc.VectorSubcoreMesh(core_axis_name="c", subcore_axis_name="s")
    run = functools.partial(
        pl.kernel,
        mesh=mesh,
        out_type=jax.ShapeDtypeStruct((_ROWS, D_MODEL), jnp.float32),
        scratch_types=(
            [pltpu.VMEM((BATCH, _STRIPE), jnp.int32)]
            + [pltpu.VMEM((_CHUNK, _PWORDS), jnp.int32)] * 2
            + [pltpu.VMEM((_CHUNK, D_MODEL), jnp.float32)] * _NBUF
            + [pltpu.SemaphoreType.DMA] * (5 * _NBUF + 3)
        ),
    )(_sc_body)
    return run(tok_table, idx, pos_packed)


def kernel(x, tok_table):
    idx = x.astype(jnp.int32)
    pos_packed = jnp.asarray(_POS_PACKED)
    out = _embed(idx, tok_table, pos_packed)
    return out.reshape(BATCH, SEQ_LEN, D_MODEL)
